# baseline (speedup 1.0000x reference)
import functools

import jax
import jax.numpy as jnp
from jax import lax
from jax.experimental import pallas as pl
from jax.experimental.pallas import tpu as pltpu
from jax.experimental.pallas import tpu_sc as plsc

NF = 128
N = 10129
NP = 10240
E = 320000
NC, NS = 2, 16
NW = NC * NS
CH = 96
NCH = 2 * (-(-E // (NW * CH * 2)))
NPAIR = NCH // 2
EPW = NCH * CH
EP = NW * EPW
RPS = NP // NS
BN = 1024
NBLK = NP // BN


def _mesh():
    return plsc.VectorSubcoreMesh(
        core_axis_name="c", subcore_axis_name="s", num_cores=NC, num_subcores=NS
    )


def _sc_degree():
    def body(dst_hbm, ones_hbm, z_hbm, out_hbm, dst_v, ones_v, acc):
        c = lax.axis_index("c")
        s = lax.axis_index("s")
        w = s * NC + c
        pltpu.sync_copy(z_hbm, acc.at[pl.ds(s * RPS, RPS)])
        pltpu.sync_copy(ones_hbm, ones_v)
        pltpu.sync_copy(dst_hbm.at[w], dst_v)
        plsc.subcore_barrier()

        def step(i, carry):
            pltpu.sync_copy(ones_v, acc.at[dst_v.at[i]], add=True)
            return carry

        lax.fori_loop(0, NCH, step, 0)
        plsc.subcore_barrier()
        base = c * NP + s * RPS
        pltpu.sync_copy(acc.at[pl.ds(s * RPS, RPS)], out_hbm.at[pl.ds(base, RPS)])

    return pl.kernel(
        body,
        out_type=jax.ShapeDtypeStruct((NC * NP, 16), jnp.float32),
        mesh=_mesh(),
        compiler_params=pltpu.CompilerParams(use_tc_tiling_on_sc=False),
        scratch_types=[
            pltpu.VMEM((NCH, CH), jnp.int32),
            pltpu.VMEM((CH, 16), jnp.float32),
            pltpu.VMEM_SHARED((NP, 16), jnp.float32),
        ],
    )


def _sc_aggregate(F, ch, k0, k1):
    km = max(k0, k1)

    def body(g_hbm, src_hbm, dst_hbm, z_hbm, out_hbm, src_v, dst_v,
             rows0, rows1, acc, sem0, sem1):
        c = lax.axis_index("c")
        s = lax.axis_index("s")
        pltpu.sync_copy(z_hbm, acc.at[pl.ds(s * RPS, RPS)])

        @pl.when(c == 0)
        def _():
            pltpu.sync_copy(src_hbm.at[pl.ds(s * 2 * k0, 2 * k0)],
                            src_v.at[pl.ds(0, 2 * k0)])
            pltpu.sync_copy(dst_hbm.at[pl.ds(s * 2 * k0, 2 * k0)],
                            dst_v.at[pl.ds(0, 2 * k0)])

        @pl.when(c == 1)
        def _():
            pltpu.sync_copy(src_hbm.at[pl.ds(NS * 2 * k0 + s * 2 * k1, 2 * k1)],
                            src_v.at[pl.ds(0, 2 * k1)])
            pltpu.sync_copy(dst_hbm.at[pl.ds(NS * 2 * k0 + s * 2 * k1, 2 * k1)],
                            dst_v.at[pl.ds(0, 2 * k1)])

        plsc.subcore_barrier()
        npair = jnp.where(c == 0, k0, k1)

        pltpu.async_copy(g_hbm.at[src_v.at[0]], rows0, sem0)

        def pair(j, carry):
            @pl.when(j < npair)
            def _():
                i0 = j * 2
                i1 = i0 + 1
                pltpu.async_copy(g_hbm.at[src_v.at[i1]], rows1, sem1)
                pltpu.make_async_copy(g_hbm.at[src_v.at[i0]], rows0, sem0).wait()
                pltpu.sync_copy(rows0, acc.at[dst_v.at[i0]], add=True)

                @pl.when(j < npair - 1)
                def _():
                    pltpu.async_copy(g_hbm.at[src_v.at[i0 + 2]], rows0, sem0)

                pltpu.make_async_copy(g_hbm.at[src_v.at[i1]], rows1, sem1).wait()
                pltpu.sync_copy(rows1, acc.at[dst_v.at[i1]], add=True)

            return carry

        lax.fori_loop(0, km, pair, 0)
        plsc.subcore_barrier()
        base = c * NP + s * RPS
        pltpu.sync_copy(acc.at[pl.ds(s * RPS, RPS)], out_hbm.at[pl.ds(base, RPS)])

    return pl.kernel(
        body,
        out_type=jax.ShapeDtypeStruct((NC * NP, F), jnp.float32),
        mesh=_mesh(),
        compiler_params=pltpu.CompilerParams(use_tc_tiling_on_sc=False),
        scratch_types=[
            pltpu.VMEM((2 * km, ch), jnp.int32),
            pltpu.VMEM((2 * km, ch), jnp.int32),
            pltpu.VMEM((ch, F), jnp.float32),
            pltpu.VMEM((ch, F), jnp.float32),
            pltpu.VMEM_SHARED((NP, F), jnp.float32),
            pltpu.SemaphoreType.DMA,
            pltpu.SemaphoreType.DMA,
        ],
    )


def _sc_aggregate4(F, ch, k0, k1):
    km = max(k0, k1)

    def body(g_hbm, src_hbm, dst_hbm, z_hbm, out_hbm, src_v, dst_v,
             r0, r1, r2, r3, acc, g0, g1, g2, g3, s0, s1, s2, s3):
        rows = [r0, r1, r2, r3]
        gsem = [g0, g1, g2, g3]
        ssem = [s0, s1, s2, s3]
        c = lax.axis_index("c")
        s = lax.axis_index("s")
        pltpu.sync_copy(z_hbm, acc.at[pl.ds(s * RPS, RPS)])

        @pl.when(c == 0)
        def _():
            pltpu.sync_copy(src_hbm.at[pl.ds(s * 2 * k0, 2 * k0)],
                            src_v.at[pl.ds(0, 2 * k0)])
            pltpu.sync_copy(dst_hbm.at[pl.ds(s * 2 * k0, 2 * k0)],
                            dst_v.at[pl.ds(0, 2 * k0)])

        @pl.when(c == 1)
        def _():
            pltpu.sync_copy(src_hbm.at[pl.ds(NS * 2 * k0 + s * 2 * k1, 2 * k1)],
                            src_v.at[pl.ds(0, 2 * k1)])
            pltpu.sync_copy(dst_hbm.at[pl.ds(NS * 2 * k0 + s * 2 * k1, 2 * k1)],
                            dst_v.at[pl.ds(0, 2 * k1)])

        plsc.subcore_barrier()
        nch = 2 * jnp.where(c == 0, k0, k1)

        pltpu.async_copy(g_hbm.at[src_v.at[0]], rows[0], gsem[0])
        pltpu.async_copy(g_hbm.at[src_v.at[1]], rows[1], gsem[1])

        def group(q, carry):
            for b in range(4):
                j = q * 4 + b

                @pl.when(j < nch)
                def _(b=b, j=j):
                    bg = (b + 2) % 4

                    @pl.when(j + 2 < nch)
                    def _(b=b, j=j, bg=bg):
                        @pl.when(j >= 2)
                        def _(b=b, j=j, bg=bg):
                            pltpu.make_async_copy(
                                rows[bg], acc.at[dst_v.at[0]], ssem[bg]).wait()

                        pltpu.async_copy(g_hbm.at[src_v.at[j + 2]],
                                         rows[bg], gsem[bg])

                    pltpu.make_async_copy(g_hbm.at[src_v.at[j]],
                                          rows[b], gsem[b]).wait()
                    pltpu.async_copy(rows[b], acc.at[dst_v.at[j]],
                                     ssem[b], add=True)

            return carry

        lax.fori_loop(0, (2 * km + 3) // 4, group, 0)
        for b in range(4):
            pltpu.make_async_copy(rows[b], acc.at[dst_v.at[0]], ssem[b]).wait()
        plsc.subcore_barrier()
        base = c * NP + s * RPS
        pltpu.sync_copy(acc.at[pl.ds(s * RPS, RPS)], out_hbm.at[pl.ds(base, RPS)])

    return pl.kernel(
        body,
        out_type=jax.ShapeDtypeStruct((NC * NP, F), jnp.float32),
        mesh=_mesh(),
        compiler_params=pltpu.CompilerParams(use_tc_tiling_on_sc=False),
        scratch_types=[
            pltpu.VMEM((2 * km, ch), jnp.int32),
            pltpu.VMEM((2 * km, ch), jnp.int32),
            pltpu.VMEM((ch, F), jnp.float32),
            pltpu.VMEM((ch, F), jnp.float32),
            pltpu.VMEM((ch, F), jnp.float32),
            pltpu.VMEM((ch, F), jnp.float32),
            pltpu.VMEM_SHARED((NP, F), jnp.float32),
            pltpu.SemaphoreType.DMA,
            pltpu.SemaphoreType.DMA,
            pltpu.SemaphoreType.DMA,
            pltpu.SemaphoreType.DMA,
            pltpu.SemaphoreType.DMA,
            pltpu.SemaphoreType.DMA,
            pltpu.SemaphoreType.DMA,
            pltpu.SemaphoreType.DMA,
        ],
    )


def _dinv(d0, d1):
    return lax.rsqrt(d0[:, :1] + d1[:, :1] + 1.0)


def _leaky(v):
    return jnp.where(v >= 0.0, v, 0.01 * v)


def _tc_first(x_ref, d0_ref, d1_ref, w_ref, o_ref):
    dinv = _dinv(d0_ref[...], d1_ref[...])
    hw = jnp.dot(x_ref[...], w_ref[...], preferred_element_type=jnp.float32)
    o_ref[...] = dinv * hw


def _tc_mid(s0_ref, s1_ref, g_ref, d0_ref, d1_ref, b_ref, w_ref, o_ref):
    dinv = _dinv(d0_ref[...], d1_ref[...])
    agg = s0_ref[...] + s1_ref[...] + g_ref[...]
    h = _leaky(dinv * agg + b_ref[...])
    o_ref[...] = dinv * jnp.dot(h, w_ref[...], preferred_element_type=jnp.float32)


def _tc_head(s0_ref, s1_ref, g_ref, d0_ref, d1_ref, b_ref, wm1_ref, bm1_ref,
             wm2_ref, bm2_ref, o_ref):
    dinv = _dinv(d0_ref[...], d1_ref[...])
    out3 = dinv * (s0_ref[...] + s1_ref[...] + g_ref[...]) + b_ref[...]
    h = _leaky(jnp.dot(out3, wm1_ref[...], preferred_element_type=jnp.float32)
               + bm1_ref[...])
    logits = jnp.dot(h, wm2_ref[...], preferred_element_type=jnp.float32) + bm2_ref[...]
    rows = pl.program_id(0) * BN + lax.broadcasted_iota(jnp.int32, (BN, 4), 0)
    cols = lax.broadcasted_iota(jnp.int32, (BN, 4), 1)
    keep = (rows < 10000) | (cols == 1) | (cols == 3)
    o_ref[...] = jnp.where(keep, logits, -100.0)


def _row_spec(F):
    return pl.BlockSpec((BN, F), lambda i: (i, 0))


def _pair_specs(F):
    return (pl.BlockSpec((BN, F), lambda i: (i, 0)),
            pl.BlockSpec((BN, F), lambda i: (i + NBLK, 0)))


def _full_spec(shape):
    return pl.BlockSpec(shape, lambda i: (0, 0))


def kernel(x, edge_index, W1, b1, W2, b2, W3, b3, Wm1, bm1, Wm2, bm2):
    f32 = jnp.float32
    cand = jnp.concatenate([jnp.zeros((1, NF), f32), jnp.eye(NF, dtype=f32)], axis=0)
    xc = jnp.concatenate([x, cand, jnp.zeros((NP - N, NF), f32)], axis=0)
    pad = jnp.full((EP - E,), NP - 1, jnp.int32)
    srcp = jnp.concatenate([edge_index[0], pad]).reshape(NW, NCH, CH)
    dstp = jnp.concatenate([edge_index[1], pad]).reshape(NW, NCH, CH)
    ones16 = jnp.ones((CH, 16), f32)
    z16 = jnp.zeros((RPS, 16), f32)

    def edges_for(ch, k0, k1):
        epl = NS * 2 * (k0 + k1) * ch
        p = jnp.full((epl - E,), NP - 1, jnp.int32)
        return (jnp.concatenate([edge_index[0], p]).reshape(epl // ch, ch),
                jnp.concatenate([edge_index[1], p]).reshape(epl // ch, ch))

    L1 = (128, 54, 26)
    L2 = (128, 66, 14)
    L3 = (80, 59, 66)
    src1, dst1 = edges_for(*L1)
    src2, dst2 = edges_for(*L2)
    src3, dst3 = edges_for(*L3)

    dp0, dp1 = _pair_specs(16)

    deg = _sc_degree()(dstp, ones16, z16)

    g1 = pl.pallas_call(
        _tc_first,
        grid=(NBLK,),
        in_specs=[_row_spec(NF), dp0, dp1, _full_spec((NF, 32))],
        out_specs=_row_spec(32),
        out_shape=jax.ShapeDtypeStruct((NP, 32), f32),
    )(xc, deg, deg, W1)

    s1 = _sc_aggregate4(32, *L1)(g1, src1, dst1, jnp.zeros((RPS, 32), f32))

    sp0, sp1 = _pair_specs(32)
    g2 = pl.pallas_call(
        _tc_mid,
        grid=(NBLK,),
        in_specs=[sp0, sp1, _row_spec(32), dp0, dp1, _full_spec((1, 32)), _full_spec((32, 64))],
        out_specs=_row_spec(64),
        out_shape=jax.ShapeDtypeStruct((NP, 64), f32),
    )(s1, s1, g1, deg, deg, b1.reshape(1, 32), W2)

    s2 = _sc_aggregate4(64, *L2)(g2, src2, dst2, jnp.zeros((RPS, 64), f32))

    sp0, sp1 = _pair_specs(64)
    g3 = pl.pallas_call(
        _tc_mid,
        grid=(NBLK,),
        in_specs=[sp0, sp1, _row_spec(64), dp0, dp1, _full_spec((1, 64)), _full_spec((64, NF))],
        out_specs=_row_spec(NF),
        out_shape=jax.ShapeDtypeStruct((NP, NF), f32),
    )(s2, s2, g2, deg, deg, b2.reshape(1, 64), W3)

    s3 = _sc_aggregate(NF, *L3)(g3, src3, dst3, jnp.zeros((RPS, NF), f32))

    sp0, sp1 = _pair_specs(NF)
    out4 = pl.pallas_call(
        _tc_head,
        grid=(NBLK,),
        in_specs=[sp0, sp1, _row_spec(NF), dp0, dp1,
                  _full_spec((1, NF)), _full_spec((NF, NF)), _full_spec((1, NF)),
                  _full_spec((NF, 4)), _full_spec((1, 4))],
        out_specs=_row_spec(4),
        out_shape=jax.ShapeDtypeStruct((NP, 4), f32),
    )(s3, s3, g3, deg, deg, b3.reshape(1, NF), Wm1, bm1.reshape(1, NF),
      Wm2, bm2.reshape(1, 4))

    return (out4[:N, 0], out4[:N, 1], out4[:N, 2], out4[:N, 3])


# speedup vs baseline: 1.3514x; 1.3514x over previous
import functools

import jax
import jax.numpy as jnp
from jax import lax
from jax.experimental import pallas as pl
from jax.experimental.pallas import tpu as pltpu
from jax.experimental.pallas import tpu_sc as plsc

NF = 128
N = 10129
NP = 10240
E = 320000
NC, NS = 2, 16
NW = NC * NS
CH = 96
NCH = 2 * (-(-E // (NW * CH * 2)))
NPAIR = NCH // 2
EPW = NCH * CH
EP = NW * EPW
RPS = NP // NS
BN = 1024
NBLK = NP // BN


def _mesh():
    return plsc.VectorSubcoreMesh(
        core_axis_name="c", subcore_axis_name="s", num_cores=NC, num_subcores=NS
    )


def _sc_degree():
    def body(dst_hbm, ones_hbm, z_hbm, out_hbm, dst_v, ones_v, acc):
        c = lax.axis_index("c")
        s = lax.axis_index("s")
        w = s * NC + c
        pltpu.sync_copy(z_hbm, acc.at[pl.ds(s * RPS, RPS)])
        pltpu.sync_copy(ones_hbm, ones_v)
        pltpu.sync_copy(dst_hbm.at[w], dst_v)
        plsc.subcore_barrier()

        def step(i, carry):
            pltpu.sync_copy(ones_v, acc.at[dst_v.at[i]], add=True)
            return carry

        lax.fori_loop(0, NCH, step, 0)
        plsc.subcore_barrier()
        base = c * NP + s * RPS
        pltpu.sync_copy(acc.at[pl.ds(s * RPS, RPS)], out_hbm.at[pl.ds(base, RPS)])

    return pl.kernel(
        body,
        out_type=jax.ShapeDtypeStruct((NC * NP, 16), jnp.float32),
        mesh=_mesh(),
        compiler_params=pltpu.CompilerParams(use_tc_tiling_on_sc=False),
        scratch_types=[
            pltpu.VMEM((NCH, CH), jnp.int32),
            pltpu.VMEM((CH, 16), jnp.float32),
            pltpu.VMEM_SHARED((NP, 16), jnp.float32),
        ],
    )


def _sc_aggregate(F, ch, k0, k1):
    km = max(k0, k1)

    def body(g_hbm, src_hbm, dst_hbm, z_hbm, out_hbm, src_v, dst_v,
             rows0, rows1, acc, sem0, sem1):
        c = lax.axis_index("c")
        s = lax.axis_index("s")
        pltpu.sync_copy(z_hbm, acc.at[pl.ds(s * RPS, RPS)])

        @pl.when(c == 0)
        def _():
            pltpu.sync_copy(src_hbm.at[pl.ds(s * 2 * k0, 2 * k0)],
                            src_v.at[pl.ds(0, 2 * k0)])
            pltpu.sync_copy(dst_hbm.at[pl.ds(s * 2 * k0, 2 * k0)],
                            dst_v.at[pl.ds(0, 2 * k0)])

        @pl.when(c == 1)
        def _():
            pltpu.sync_copy(src_hbm.at[pl.ds(NS * 2 * k0 + s * 2 * k1, 2 * k1)],
                            src_v.at[pl.ds(0, 2 * k1)])
            pltpu.sync_copy(dst_hbm.at[pl.ds(NS * 2 * k0 + s * 2 * k1, 2 * k1)],
                            dst_v.at[pl.ds(0, 2 * k1)])

        plsc.subcore_barrier()
        npair = jnp.where(c == 0, k0, k1)

        pltpu.async_copy(g_hbm.at[src_v.at[0]], rows0, sem0)

        def pair(j, carry):
            @pl.when(j < npair)
            def _():
                i0 = j * 2
                i1 = i0 + 1
                pltpu.async_copy(g_hbm.at[src_v.at[i1]], rows1, sem1)
                pltpu.make_async_copy(g_hbm.at[src_v.at[i0]], rows0, sem0).wait()
                pltpu.sync_copy(rows0, acc.at[dst_v.at[i0]], add=True)

                @pl.when(j < npair - 1)
                def _():
                    pltpu.async_copy(g_hbm.at[src_v.at[i0 + 2]], rows0, sem0)

                pltpu.make_async_copy(g_hbm.at[src_v.at[i1]], rows1, sem1).wait()
                pltpu.sync_copy(rows1, acc.at[dst_v.at[i1]], add=True)

            return carry

        lax.fori_loop(0, km, pair, 0)
        plsc.subcore_barrier()
        base = c * NP + s * RPS
        pltpu.sync_copy(acc.at[pl.ds(s * RPS, RPS)], out_hbm.at[pl.ds(base, RPS)])

    return pl.kernel(
        body,
        out_type=jax.ShapeDtypeStruct((NC * NP, F), jnp.float32),
        mesh=_mesh(),
        compiler_params=pltpu.CompilerParams(use_tc_tiling_on_sc=False),
        scratch_types=[
            pltpu.VMEM((2 * km, ch), jnp.int32),
            pltpu.VMEM((2 * km, ch), jnp.int32),
            pltpu.VMEM((ch, F), jnp.float32),
            pltpu.VMEM((ch, F), jnp.float32),
            pltpu.VMEM_SHARED((NP, F), jnp.float32),
            pltpu.SemaphoreType.DMA,
            pltpu.SemaphoreType.DMA,
        ],
    )


def _dinv(d0, d1):
    return lax.rsqrt(d0[:, :1] + d1[:, :1] + 1.0)


def _leaky(v):
    return jnp.where(v >= 0.0, v, 0.01 * v)


def _tc_first(x_ref, d0_ref, d1_ref, w_ref, o_ref):
    dinv = _dinv(d0_ref[...], d1_ref[...])
    hw = jnp.dot(x_ref[...], w_ref[...], preferred_element_type=jnp.float32)
    o_ref[...] = dinv * hw


def _tc_mid(s0_ref, s1_ref, g_ref, d0_ref, d1_ref, b_ref, w_ref, o_ref):
    dinv = _dinv(d0_ref[...], d1_ref[...])
    agg = s0_ref[...] + s1_ref[...] + g_ref[...]
    h = _leaky(dinv * agg + b_ref[...])
    o_ref[...] = dinv * jnp.dot(h, w_ref[...], preferred_element_type=jnp.float32)


def _tc_head(s0_ref, s1_ref, g_ref, d0_ref, d1_ref, b_ref, wm1_ref, bm1_ref,
             wm2_ref, bm2_ref, o_ref):
    dinv = _dinv(d0_ref[...], d1_ref[...])
    out3 = dinv * (s0_ref[...] + s1_ref[...] + g_ref[...]) + b_ref[...]
    h = _leaky(jnp.dot(out3, wm1_ref[...], preferred_element_type=jnp.float32)
               + bm1_ref[...])
    logits = jnp.dot(h, wm2_ref[...], preferred_element_type=jnp.float32) + bm2_ref[...]
    rows = pl.program_id(0) * BN + lax.broadcasted_iota(jnp.int32, (BN, 4), 0)
    cols = lax.broadcasted_iota(jnp.int32, (BN, 4), 1)
    keep = (rows < 10000) | (cols == 1) | (cols == 3)
    o_ref[...] = jnp.where(keep, logits, -100.0)


def _row_spec(F):
    return pl.BlockSpec((BN, F), lambda i: (i, 0))


def _pair_specs(F):
    return (pl.BlockSpec((BN, F), lambda i: (i, 0)),
            pl.BlockSpec((BN, F), lambda i: (i + NBLK, 0)))


def _full_spec(shape):
    return pl.BlockSpec(shape, lambda i: (0, 0))


def kernel(x, edge_index, W1, b1, W2, b2, W3, b3, Wm1, bm1, Wm2, bm2):
    f32 = jnp.float32
    cand = jnp.concatenate([jnp.zeros((1, NF), f32), jnp.eye(NF, dtype=f32)], axis=0)
    xc = jnp.concatenate([x, cand, jnp.zeros((NP - N, NF), f32)], axis=0)
    pad = jnp.full((EP - E,), NP - 1, jnp.int32)
    srcp = jnp.concatenate([edge_index[0], pad]).reshape(NW, NCH, CH)
    dstp = jnp.concatenate([edge_index[1], pad]).reshape(NW, NCH, CH)
    ones16 = jnp.ones((CH, 16), f32)
    z16 = jnp.zeros((RPS, 16), f32)

    def edges_for(ch, k0, k1):
        epl = NS * 2 * (k0 + k1) * ch
        p = jnp.full((epl - E,), NP - 1, jnp.int32)
        return (jnp.concatenate([edge_index[0], p]).reshape(epl // ch, ch),
                jnp.concatenate([edge_index[1], p]).reshape(epl // ch, ch))

    L1 = (128, 53, 26)
    L2 = (128, 66, 13)
    L3 = (80, 62, 63)
    src1, dst1 = edges_for(*L1)
    src2, dst2 = edges_for(*L2)
    src3, dst3 = edges_for(*L3)

    dp0, dp1 = _pair_specs(16)

    deg = _sc_degree()(dstp, ones16, z16)

    g1 = pl.pallas_call(
        _tc_first,
        grid=(NBLK,),
        in_specs=[_row_spec(NF), dp0, dp1, _full_spec((NF, 32))],
        out_specs=_row_spec(32),
        out_shape=jax.ShapeDtypeStruct((NP, 32), f32),
    )(xc, deg, deg, W1)

    s1 = _sc_aggregate(32, *L1)(g1, src1, dst1, jnp.zeros((RPS, 32), f32))

    sp0, sp1 = _pair_specs(32)
    g2 = pl.pallas_call(
        _tc_mid,
        grid=(NBLK,),
        in_specs=[sp0, sp1, _row_spec(32), dp0, dp1, _full_spec((1, 32)), _full_spec((32, 64))],
        out_specs=_row_spec(64),
        out_shape=jax.ShapeDtypeStruct((NP, 64), f32),
    )(s1, s1, g1, deg, deg, b1.reshape(1, 32), W2)

    s2 = _sc_aggregate(64, *L2)(g2, src2, dst2, jnp.zeros((RPS, 64), f32))

    sp0, sp1 = _pair_specs(64)
    g3 = pl.pallas_call(
        _tc_mid,
        grid=(NBLK,),
        in_specs=[sp0, sp1, _row_spec(64), dp0, dp1, _full_spec((1, 64)), _full_spec((64, NF))],
        out_specs=_row_spec(NF),
        out_shape=jax.ShapeDtypeStruct((NP, NF), f32),
    )(s2, s2, g2, deg, deg, b2.reshape(1, 64), W3)

    s3 = _sc_aggregate(NF, *L3)(g3, src3, dst3, jnp.zeros((RPS, NF), f32))

    sp0, sp1 = _pair_specs(NF)
    out4 = pl.pallas_call(
        _tc_head,
        grid=(NBLK,),
        in_specs=[sp0, sp1, _row_spec(NF), dp0, dp1,
                  _full_spec((1, NF)), _full_spec((NF, NF)), _full_spec((1, NF)),
                  _full_spec((NF, 4)), _full_spec((1, 4))],
        out_specs=_row_spec(4),
        out_shape=jax.ShapeDtypeStruct((NP, 4), f32),
    )(s3, s3, g3, deg, deg, b3.reshape(1, NF), Wm1, bm1.reshape(1, NF),
      Wm2, bm2.reshape(1, 4))

    return (out4[:N, 0], out4[:N, 1], out4[:N, 2], out4[:N, 3])


# speedup vs baseline: 1.3519x; 1.0003x over previous
import jax
import jax.numpy as jnp
from jax import lax
from jax.experimental import pallas as pl
from jax.experimental.pallas import tpu as pltpu
from jax.experimental.pallas import tpu_sc as plsc

NF = 128
N = 10129
NP = 10240
E = 320000
NC, NS = 2, 16
NW = NC * NS
CH = 96
NCH = 2 * (-(-E // (NW * CH * 2)))
NPAIR = NCH // 2
EPW = NCH * CH
EP = NW * EPW
RPS = NP // NS
BN = 1024
NBLK = NP // BN


def _mesh():
    return plsc.VectorSubcoreMesh(
        core_axis_name="c", subcore_axis_name="s", num_cores=NC, num_subcores=NS
    )


def _sc_degree():
    def body(dst_hbm, ones_hbm, z_hbm, out_hbm, dst_v, ones_v, acc):
        c = lax.axis_index("c")
        s = lax.axis_index("s")
        w = s * NC + c
        pltpu.sync_copy(z_hbm, acc.at[pl.ds(s * RPS, RPS)])
        pltpu.sync_copy(ones_hbm, ones_v)
        pltpu.sync_copy(dst_hbm.at[w], dst_v)
        plsc.subcore_barrier()

        def step(i, carry):
            pltpu.sync_copy(ones_v, acc.at[dst_v.at[i]], add=True)
            return carry

        lax.fori_loop(0, NCH, step, 0)
        plsc.subcore_barrier()
        base = c * NP + s * RPS
        pltpu.sync_copy(acc.at[pl.ds(s * RPS, RPS)], out_hbm.at[pl.ds(base, RPS)])

    return pl.kernel(
        body,
        out_type=jax.ShapeDtypeStruct((NC * NP, 16), jnp.float32),
        mesh=_mesh(),
        compiler_params=pltpu.CompilerParams(use_tc_tiling_on_sc=False),
        scratch_types=[
            pltpu.VMEM((NCH, CH), jnp.int32),
            pltpu.VMEM((CH, 16), jnp.float32),
            pltpu.VMEM_SHARED((NP, 16), jnp.float32),
        ],
    )


def _sc_aggregate(F, ch, k0, k1):
    km = max(k0, k1)

    def body(g_hbm, src_hbm, dst_hbm, z_hbm, out_hbm, src_v, dst_v,
             rows0, rows1, acc, sem0, sem1):
        c = lax.axis_index("c")
        s = lax.axis_index("s")
        pltpu.sync_copy(z_hbm, acc.at[pl.ds(s * RPS, RPS)])

        @pl.when(c == 0)
        def _():
            pltpu.sync_copy(src_hbm.at[pl.ds(s * 2 * k0, 2 * k0)],
                            src_v.at[pl.ds(0, 2 * k0)])
            pltpu.sync_copy(dst_hbm.at[pl.ds(s * 2 * k0, 2 * k0)],
                            dst_v.at[pl.ds(0, 2 * k0)])

        @pl.when(c == 1)
        def _():
            pltpu.sync_copy(src_hbm.at[pl.ds(NS * 2 * k0 + s * 2 * k1, 2 * k1)],
                            src_v.at[pl.ds(0, 2 * k1)])
            pltpu.sync_copy(dst_hbm.at[pl.ds(NS * 2 * k0 + s * 2 * k1, 2 * k1)],
                            dst_v.at[pl.ds(0, 2 * k1)])

        plsc.subcore_barrier()
        npair = jnp.where(c == 0, k0, k1)

        pltpu.async_copy(g_hbm.at[src_v.at[0]], rows0, sem0)

        def pair(j, carry):
            @pl.when(j < npair)
            def _():
                i0 = j * 2
                i1 = i0 + 1
                pltpu.async_copy(g_hbm.at[src_v.at[i1]], rows1, sem1)
                pltpu.make_async_copy(g_hbm.at[src_v.at[i0]], rows0, sem0).wait()
                pltpu.sync_copy(rows0, acc.at[dst_v.at[i0]], add=True)

                @pl.when(j < npair - 1)
                def _():
                    pltpu.async_copy(g_hbm.at[src_v.at[i0 + 2]], rows0, sem0)

                pltpu.make_async_copy(g_hbm.at[src_v.at[i1]], rows1, sem1).wait()
                pltpu.sync_copy(rows1, acc.at[dst_v.at[i1]], add=True)

            return carry

        lax.fori_loop(0, km, pair, 0)
        plsc.subcore_barrier()
        base = c * NP + s * RPS
        pltpu.sync_copy(acc.at[pl.ds(s * RPS, RPS)], out_hbm.at[pl.ds(base, RPS)])

    return pl.kernel(
        body,
        out_type=jax.ShapeDtypeStruct((NC * NP, F), jnp.float32),
        mesh=_mesh(),
        compiler_params=pltpu.CompilerParams(use_tc_tiling_on_sc=False),
        scratch_types=[
            pltpu.VMEM((2 * km, ch), jnp.int32),
            pltpu.VMEM((2 * km, ch), jnp.int32),
            pltpu.VMEM((ch, F), jnp.float32),
            pltpu.VMEM((ch, F), jnp.float32),
            pltpu.VMEM_SHARED((NP, F), jnp.float32),
            pltpu.SemaphoreType.DMA,
            pltpu.SemaphoreType.DMA,
        ],
    )


def _dinv(d0, d1):
    return lax.rsqrt(d0[:, :1] + d1[:, :1] + 1.0)


def _leaky(v):
    return jnp.where(v >= 0.0, v, 0.01 * v)


def _tc_first(x_ref, d0_ref, d1_ref, w_ref, o_ref):
    dinv = _dinv(d0_ref[...], d1_ref[...])
    hw = jnp.dot(x_ref[...], w_ref[...], preferred_element_type=jnp.float32)
    o_ref[...] = dinv * hw


def _tc_mid(s0_ref, s1_ref, g_ref, d0_ref, d1_ref, b_ref, w_ref, o_ref):
    dinv = _dinv(d0_ref[...], d1_ref[...])
    agg = s0_ref[...] + s1_ref[...] + g_ref[...]
    h = _leaky(dinv * agg + b_ref[...])
    o_ref[...] = dinv * jnp.dot(h, w_ref[...], preferred_element_type=jnp.float32)


def _tc_head(s0_ref, s1_ref, g_ref, d0_ref, d1_ref, b_ref, wm1_ref, bm1_ref,
             wm2_ref, bm2_ref, o_ref):
    dinv = _dinv(d0_ref[...], d1_ref[...])
    out3 = dinv * (s0_ref[...] + s1_ref[...] + g_ref[...]) + b_ref[...]
    h = _leaky(jnp.dot(out3, wm1_ref[...], preferred_element_type=jnp.float32)
               + bm1_ref[...])
    logits = jnp.dot(h, wm2_ref[...], preferred_element_type=jnp.float32) + bm2_ref[...]
    rows = pl.program_id(0) * BN + lax.broadcasted_iota(jnp.int32, (BN, 4), 0)
    cols = lax.broadcasted_iota(jnp.int32, (BN, 4), 1)
    keep = (rows < 10000) | (cols == 1) | (cols == 3)
    o_ref[...] = jnp.where(keep, logits, -100.0)


def _row_spec(F):
    return pl.BlockSpec((BN, F), lambda i: (i, 0))


def _pair_specs(F):
    return (pl.BlockSpec((BN, F), lambda i: (i, 0)),
            pl.BlockSpec((BN, F), lambda i: (i + NBLK, 0)))


def _full_spec(shape):
    return pl.BlockSpec(shape, lambda i: (0, 0))


def kernel(x, edge_index, W1, b1, W2, b2, W3, b3, Wm1, bm1, Wm2, bm2):
    f32 = jnp.float32
    cand = jnp.concatenate([jnp.zeros((1, NF), f32), jnp.eye(NF, dtype=f32)], axis=0)
    xc = jnp.concatenate([x, cand, jnp.zeros((NP - N, NF), f32)], axis=0)
    pad = jnp.full((EP - E,), NP - 1, jnp.int32)
    srcp = jnp.concatenate([edge_index[0], pad]).reshape(NW, NCH, CH)
    dstp = jnp.concatenate([edge_index[1], pad]).reshape(NW, NCH, CH)
    ones16 = jnp.ones((CH, 16), f32)
    z16 = jnp.zeros((RPS, 16), f32)

    def edges_for(ch, k0, k1):
        epl = NS * 2 * (k0 + k1) * ch
        p = jnp.full((epl - E,), NP - 1, jnp.int32)
        return (jnp.concatenate([edge_index[0], p]).reshape(epl // ch, ch),
                jnp.concatenate([edge_index[1], p]).reshape(epl // ch, ch))

    L1 = (128, 53, 26)
    L2 = (128, 66, 13)
    L3 = (80, 62, 63)
    src1, dst1 = edges_for(*L1)
    src2, dst2 = edges_for(*L2)
    src3, dst3 = edges_for(*L3)

    dp0, dp1 = _pair_specs(16)

    deg = _sc_degree()(dstp, ones16, z16)

    g1 = pl.pallas_call(
        _tc_first,
        grid=(NBLK,),
        in_specs=[_row_spec(NF), dp0, dp1, _full_spec((NF, 32))],
        out_specs=_row_spec(32),
        out_shape=jax.ShapeDtypeStruct((NP, 32), f32),
    )(xc, deg, deg, W1)

    s1 = _sc_aggregate(32, *L1)(g1, src1, dst1, jnp.zeros((RPS, 32), f32))

    sp0, sp1 = _pair_specs(32)
    g2 = pl.pallas_call(
        _tc_mid,
        grid=(NBLK,),
        in_specs=[sp0, sp1, _row_spec(32), dp0, dp1, _full_spec((1, 32)), _full_spec((32, 64))],
        out_specs=_row_spec(64),
        out_shape=jax.ShapeDtypeStruct((NP, 64), f32),
    )(s1, s1, g1, deg, deg, b1.reshape(1, 32), W2)

    s2 = _sc_aggregate(64, *L2)(g2, src2, dst2, jnp.zeros((RPS, 64), f32))

    sp0, sp1 = _pair_specs(64)
    g3 = pl.pallas_call(
        _tc_mid,
        grid=(NBLK,),
        in_specs=[sp0, sp1, _row_spec(64), dp0, dp1, _full_spec((1, 64)), _full_spec((64, NF))],
        out_specs=_row_spec(NF),
        out_shape=jax.ShapeDtypeStruct((NP, NF), f32),
    )(s2, s2, g2, deg, deg, b2.reshape(1, 64), W3)

    s3 = _sc_aggregate(NF, *L3)(g3, src3, dst3, jnp.zeros((RPS, NF), f32))

    sp0, sp1 = _pair_specs(NF)
    out4 = pl.pallas_call(
        _tc_head,
        grid=(NBLK,),
        in_specs=[sp0, sp1, _row_spec(NF), dp0, dp1,
                  _full_spec((1, NF)), _full_spec((NF, NF)), _full_spec((1, NF)),
                  _full_spec((NF, 4)), _full_spec((1, 4))],
        out_specs=_row_spec(4),
        out_shape=jax.ShapeDtypeStruct((NP, 4), f32),
    )(s3, s3, g3, deg, deg, b3.reshape(1, NF), Wm1, bm1.reshape(1, NF),
      Wm2, bm2.reshape(1, 4))

    return (out4[:N, 0], out4[:N, 1], out4[:N, 2], out4[:N, 3])
